# baseline (speedup 1.0000x reference)
import functools

import jax
import jax.numpy as jnp
from jax import lax
from jax.experimental import pallas as pl
from jax.experimental.pallas import tpu as pltpu
from jax.experimental.pallas import tpu_sc as plsc

N = 10000
D = 128
NC = 2
NS = 16
L = 16
NW = NC * NS
F = D // NW
NP = 10240
NPC = NP // NS
CHUNK = 3200
DCH = 2000


def _rsqrt16(v):
    vi = plsc.bitcast(v, jnp.int32)
    yi = jnp.int32(0x5F3759DF) - lax.shift_right_logical(vi, 1)
    y = plsc.bitcast(yi, jnp.float32)
    for _ in range(3):
        y = y * (1.5 - 0.5 * v * y * y)
    return y


def _zero_f32(ref, n):
    z = jnp.zeros((L,), jnp.float32)

    @plsc.parallel_loop(0, n // L, 1, unroll=4)
    def _(i):
        ref[pl.ds(i * L, L)] = z


def _sc_body(xtl, srcl, dstl, ewl, xtg, srcg, dstg, ewg,
             accl, accg,
             xt0, xt1, xt2, xt3, ac0, ac1, ac2, ac3,
             dinv_v, srcb, dstb, ewb, tmp, sh_part, sh_red, sems):
    xts = (xt0, xt1, xt2, xt3)
    acs = (ac0, ac1, ac2, ac3)
    cid = lax.axis_index("c")
    sid = lax.axis_index("s")
    wid = sid * NC + cid

    def run_graph(xtH, srcH, dstH, ewH, accH):
        E = srcH.shape[0]

        _zero_f32(dinv_v, NP)
        epert = E // NS
        base = pl.multiple_of(sid * epert, DCH)

        def deg_chunk(k, _):
            off = pl.multiple_of(base + k * DCH, DCH)
            pltpu.sync_copy(dstH.at[pl.ds(off, DCH)], dstb.at[pl.ds(0, DCH)])
            pltpu.sync_copy(ewH.at[pl.ds(off, DCH)], ewb.at[pl.ds(0, DCH)])

            @plsc.parallel_loop(0, DCH // L, 1, unroll=4)
            def _(i):
                d16 = dstb[pl.ds(i * L, L)]
                w16 = ewb[pl.ds(i * L, L)]
                plsc.addupdate_scatter(dinv_v, [d16], w16)

            return 0

        lax.fori_loop(0, epert // DCH, deg_chunk, 0)

        pltpu.sync_copy(dinv_v, sh_part.at[sid])
        plsc.subcore_barrier()

        cbase = pl.multiple_of(sid * NPC, NPC)
        pltpu.sync_copy(sh_part.at[0, pl.ds(cbase, NPC)], tmp.at[pl.ds(0, NPC)])
        for t in range(1, NS):
            pltpu.sync_copy(sh_part.at[t, pl.ds(cbase, NPC)],
                            tmp.at[pl.ds(NPC, NPC)])

            @plsc.parallel_loop(0, NPC // L, 1, unroll=4)
            def _(i):
                tmp[pl.ds(i * L, L)] += tmp[pl.ds(NPC + i * L, L)]

        @plsc.parallel_loop(0, NPC // L, 1, unroll=2)
        def _(i):
            v = tmp[pl.ds(i * L, L)] + 1.0
            tmp[pl.ds(i * L, L)] = _rsqrt16(v)

        pltpu.sync_copy(tmp.at[pl.ds(0, NPC)], sh_red.at[pl.ds(cbase, NPC)])
        plsc.subcore_barrier()
        pltpu.sync_copy(sh_red, dinv_v)

        for d in range(F):
            doff = pl.multiple_of((wid * F + d) * N, N)
            pltpu.sync_copy(xtH.at[pl.ds(doff, N)], xts[d])

            xd, ad = xts[d], acs[d]

            @plsc.parallel_loop(0, N // L, 1, unroll=4)
            def _(i):
                u = xd[pl.ds(i * L, L)] * dinv_v[pl.ds(i * L, L)]
                xd[pl.ds(i * L, L)] = u
                ad[pl.ds(i * L, L)] = u

        nch = E // CHUNK

        def issue(k, slot):
            off = pl.multiple_of(k * CHUNK, CHUNK)
            sl = pl.ds(slot * CHUNK, CHUNK)
            pltpu.async_copy(srcH.at[pl.ds(off, CHUNK)], srcb.at[sl],
                             sems.at[slot])
            pltpu.async_copy(dstH.at[pl.ds(off, CHUNK)], dstb.at[sl],
                             sems.at[slot])
            pltpu.async_copy(ewH.at[pl.ds(off, CHUNK)], ewb.at[sl],
                             sems.at[slot])

        def drain(slot):
            sl = pl.ds(slot * CHUNK, CHUNK)
            pltpu.make_async_copy(srcH.at[pl.ds(0, CHUNK)], srcb.at[sl],
                                  sems.at[slot]).wait()
            pltpu.make_async_copy(dstH.at[pl.ds(0, CHUNK)], dstb.at[sl],
                                  sems.at[slot]).wait()
            pltpu.make_async_copy(ewH.at[pl.ds(0, CHUNK)], ewb.at[sl],
                                  sems.at[slot]).wait()

        def process(slot):
            @plsc.parallel_loop(0, CHUNK // L, 1, unroll=4)
            def _(i):
                s16 = srcb[pl.ds(slot * CHUNK + i * L, L)]
                d16 = dstb[pl.ds(slot * CHUNK + i * L, L)]
                w16 = ewb[pl.ds(slot * CHUNK + i * L, L)]
                for d in range(F):
                    xv = plsc.load_gather(xts[d], [s16])
                    plsc.addupdate_scatter(acs[d], [d16], xv * w16)

        issue(0, 0)

        def pair(kk, _):
            for slot in (0, 1):
                k = kk * 2 + slot

                @pl.when(k + 1 < nch)
                def _():
                    issue(k + 1, 1 - slot)

                drain(slot)
                process(slot)
            return 0

        lax.fori_loop(0, nch // 2, pair, 0)

        for d in range(F):
            ad = acs[d]

            @plsc.parallel_loop(0, N // L, 1, unroll=4)
            def _(i):
                ad[pl.ds(i * L, L)] *= dinv_v[pl.ds(i * L, L)]

            doff = pl.multiple_of((wid * F + d) * N, N)
            pltpu.sync_copy(ad, accH.at[pl.ds(doff, N)])

    run_graph(xtl, srcl, dstl, ewl, accl)
    run_graph(xtg, srcg, dstg, ewg, accg)


_sc_propagate = pl.kernel(
    _sc_body,
    out_type=[
        jax.ShapeDtypeStruct((D * N,), jnp.float32),
        jax.ShapeDtypeStruct((D * N,), jnp.float32),
    ],
    mesh=plsc.VectorSubcoreMesh(core_axis_name="c", subcore_axis_name="s"),
    compiler_params=pltpu.CompilerParams(needs_layout_passes=False),
    scratch_types=[
        pltpu.VMEM((N,), jnp.float32),
        pltpu.VMEM((N,), jnp.float32),
        pltpu.VMEM((N,), jnp.float32),
        pltpu.VMEM((N,), jnp.float32),
        pltpu.VMEM((N,), jnp.float32),
        pltpu.VMEM((N,), jnp.float32),
        pltpu.VMEM((N,), jnp.float32),
        pltpu.VMEM((N,), jnp.float32),
        pltpu.VMEM((NP,), jnp.float32),
        pltpu.VMEM((2 * CHUNK,), jnp.int32),
        pltpu.VMEM((2 * CHUNK,), jnp.int32),
        pltpu.VMEM((2 * CHUNK,), jnp.float32),
        pltpu.VMEM((2 * NPC,), jnp.float32),
        pltpu.VMEM_SHARED((NS, NP), jnp.float32),
        pltpu.VMEM_SHARED((NP,), jnp.float32),
        pltpu.SemaphoreType.DMA((2,)),
    ],
)


def _gates(yT, Wz, bz, Lz, lzb, Wh, bh, Lh, lhb):
    Lzt = Lz[...][:D, :]
    Mz = jnp.dot(Wz[...], Lzt, preferred_element_type=jnp.float32)
    czb = jnp.dot(bz[...], Lzt, preferred_element_type=jnp.float32) + lzb[...]
    Lht = Lh[...][:D, :]
    Mh = jnp.dot(Wh[...], Lht, preferred_element_type=jnp.float32)
    chb = jnp.dot(bh[...], Lht, preferred_element_type=jnp.float32) + lhb[...]
    z = lax.dot_general(yT, Mz, (((0,), (0,)), ((), ())),
                        preferred_element_type=jnp.float32)
    h = lax.dot_general(yT, Mh, (((0,), (0,)), ((), ())),
                        preferred_element_type=jnp.float32)
    return (1.0 - jax.nn.sigmoid(z + czb)) * jnp.tanh(h + chb)


def _tc_body(ytl, ytg,
             Wzl, bzl, Lzl, lzbl, Whl, bhl, Lhl, lhbl,
             Wzg, bzg, Lzg, lzbg, Whg, bhg, Lhg, lhbg,
             outl, outg):
    outl[...] = _gates(ytl[...], Wzl, bzl, Lzl, lzbl, Whl, bhl, Lhl, lhbl)
    outg[...] = _gates(ytg[...], Wzg, bzg, Lzg, lzbg, Whg, bhg, Lhg, lhbg)


_tc_combine = pl.pallas_call(
    _tc_body,
    out_shape=[jax.ShapeDtypeStruct((N, D), jnp.float32),
               jax.ShapeDtypeStruct((N, D), jnp.float32)],
)


def _param_block(params):
    Wz, bz, Lz, lzb, _Wr, _br, _Lr, _lrb, Wh, bh, Lh, lhb = params
    return (Wz, bz.reshape(1, D), Lz, lzb.reshape(1, D),
            Wh, bh.reshape(1, D), Lh, lhb.reshape(1, D))


def kernel(local_x, global_x, local_edge_index, global_edge_index,
           local_edge_weight, global_edge_weight, local_params, global_params):
    xtl = local_x.T.reshape(-1)
    xtg = global_x.T.reshape(-1)
    ytl, ytg = _sc_propagate(
        xtl, local_edge_index[0], local_edge_index[1], local_edge_weight,
        xtg, global_edge_index[0], global_edge_index[1], global_edge_weight)
    out_l, out_g = _tc_combine(
        ytl.reshape(D, N), ytg.reshape(D, N),
        *_param_block(local_params), *_param_block(global_params))
    return (out_l, out_g)


# speedup vs baseline: 1.2322x; 1.2322x over previous
import functools

import jax
import jax.numpy as jnp
from jax import lax
from jax.experimental import pallas as pl
from jax.experimental.pallas import tpu as pltpu
from jax.experimental.pallas import tpu_sc as plsc

N = 10000
D = 128
NC = 2
NS = 16
L = 16
NW = NC * NS
F = D // NW
NP = 10240
NPC = NP // NS
CHUNK = 3200
DCH = 2000


def _rsqrt16(v):
    vi = plsc.bitcast(v, jnp.int32)
    yi = jnp.int32(0x5F3759DF) - lax.shift_right_logical(vi, 1)
    y = plsc.bitcast(yi, jnp.float32)
    for _ in range(3):
        y = y * (1.5 - 0.5 * v * y * y)
    return y


def _zero_f32(ref, n):
    z = jnp.zeros((L,), jnp.float32)

    @plsc.parallel_loop(0, n // L, 1, unroll=4)
    def _(i):
        ref[pl.ds(i * L, L)] = z


def _sc_body(xtl, sdl, ewl, xtg, sdg, ewg,
             accl, accg,
             xp01, xp23, ac0, ac1, ac2, ac3,
             dinv_v, sdb, ewb, tmp, sh_part, sh_red, sems):
    xps = (xp01, xp23)
    acs = (ac0, ac1, ac2, ac3)
    cid = lax.axis_index("c")
    sid = lax.axis_index("s")
    wid = sid * NC + cid

    def run_graph(xtH, sdH, ewH, accH):
        E = sdH.shape[0]

        for d in range(F):
            doff = pl.multiple_of((wid * F + d) * N, N)
            pltpu.async_copy(xtH.at[pl.ds(doff, N)], acs[d], sems.at[2])

        _zero_f32(dinv_v, NP)
        epert = E // NS
        base = pl.multiple_of(sid * epert, DCH)

        ndch = epert // DCH

        def dissue(k, slot):
            off = pl.multiple_of(base + k * DCH, DCH)
            sl = pl.ds(slot * CHUNK, DCH)
            pltpu.async_copy(sdH.at[pl.ds(off, DCH)], sdb.at[sl],
                             sems.at[slot])
            pltpu.async_copy(ewH.at[pl.ds(off, DCH)], ewb.at[sl],
                             sems.at[slot])

        def ddrain(slot):
            sl = pl.ds(slot * CHUNK, DCH)
            pltpu.make_async_copy(sdH.at[pl.ds(0, DCH)], sdb.at[sl],
                                  sems.at[slot]).wait()
            pltpu.make_async_copy(ewH.at[pl.ds(0, DCH)], ewb.at[sl],
                                  sems.at[slot]).wait()

        dissue(0, 0)

        def deg_pair(kk, _):
            for slot in (0, 1):
                k = kk * 2 + slot

                @pl.when(k < ndch)
                def _():
                    @pl.when(k + 1 < ndch)
                    def _():
                        dissue(k + 1, 1 - slot)

                    ddrain(slot)

                    @plsc.parallel_loop(0, DCH // L, 1, unroll=4)
                    def _(i):
                        d16 = jnp.bitwise_and(
                            sdb[pl.ds(slot * CHUNK + i * L, L)], 0x3FFF)
                        w16 = ewb[pl.ds(slot * CHUNK + i * L, L)]
                        plsc.addupdate_scatter(dinv_v, [d16], w16)

            return 0

        lax.fori_loop(0, (ndch + 1) // 2, deg_pair, 0)

        for r in range(NS):
            pltpu.async_copy(
                dinv_v.at[pl.ds(r * NPC, NPC)],
                sh_part.at[pl.ds((r * NS + sid) * NPC, NPC)], sems.at[1])
        for r in range(NS):
            pltpu.make_async_copy(
                dinv_v.at[pl.ds(0, NPC)],
                sh_part.at[pl.ds(sid * NPC, NPC)], sems.at[1]).wait()
        plsc.subcore_barrier()

        spb = pl.multiple_of(sid * (NS * NPC), NS * NPC)
        pltpu.sync_copy(sh_part.at[pl.ds(spb, NS * NPC)], dinv_v)

        @plsc.parallel_loop(0, NPC // L, 1, unroll=2)
        def _(i):
            acc = dinv_v[pl.ds(i * L, L)]
            for t in range(1, NS):
                acc += dinv_v[pl.ds(t * NPC + i * L, L)]
            tmp[pl.ds(i * L, L)] = _rsqrt16(acc + 1.0)

        cbase = pl.multiple_of(sid * NPC, NPC)
        pltpu.sync_copy(tmp.at[pl.ds(0, NPC)], sh_red.at[pl.ds(cbase, NPC)])
        plsc.subcore_barrier()
        pltpu.sync_copy(sh_red, dinv_v)

        for d in range(F):
            pltpu.make_async_copy(xtH.at[pl.ds(0, N)], acs[d],
                                  sems.at[2]).wait()
        for p in range(2):
            a0, a1, xp = acs[2 * p], acs[2 * p + 1], xps[p]

            @plsc.parallel_loop(0, N // L, 1, unroll=4)
            def _(i):
                dv = dinv_v[pl.ds(i * L, L)]
                u0 = a0[pl.ds(i * L, L)] * dv
                u1 = a1[pl.ds(i * L, L)] * dv
                a0[pl.ds(i * L, L)] = u0
                a1[pl.ds(i * L, L)] = u1
                pk = plsc.pack(u0, u1, format=plsc.PackFormat.INTERLEAVED)
                xp[pl.ds(i * L, L)] = plsc.bitcast(pk, jnp.int32)

        nch = E // CHUNK

        def issue(k, slot):
            off = pl.multiple_of(k * CHUNK, CHUNK)
            sl = pl.ds(slot * CHUNK, CHUNK)
            pltpu.async_copy(sdH.at[pl.ds(off, CHUNK)], sdb.at[sl],
                             sems.at[slot])
            pltpu.async_copy(ewH.at[pl.ds(off, CHUNK)], ewb.at[sl],
                             sems.at[slot])

        def drain(slot):
            sl = pl.ds(slot * CHUNK, CHUNK)
            pltpu.make_async_copy(sdH.at[pl.ds(0, CHUNK)], sdb.at[sl],
                                  sems.at[slot]).wait()
            pltpu.make_async_copy(ewH.at[pl.ds(0, CHUNK)], ewb.at[sl],
                                  sems.at[slot]).wait()

        def process(slot):
            @plsc.parallel_loop(0, CHUNK // L, 1, unroll=4)
            def _(i):
                sd16 = sdb[pl.ds(slot * CHUNK + i * L, L)]
                s16 = lax.shift_right_logical(sd16, 14)
                d16 = jnp.bitwise_and(sd16, 0x3FFF)
                w16 = ewb[pl.ds(slot * CHUNK + i * L, L)]
                for p in range(2):
                    g = plsc.load_gather(xps[p], [s16])
                    u0, u1 = plsc.unpack(
                        plsc.bitcast(g, jnp.bfloat16),
                        format=plsc.PackFormat.INTERLEAVED)
                    plsc.addupdate_scatter(acs[2 * p], [d16], u0 * w16)
                    plsc.addupdate_scatter(acs[2 * p + 1], [d16], u1 * w16)

        issue(0, 0)

        def pair(kk, _):
            for slot in (0, 1):
                k = kk * 2 + slot

                @pl.when(k + 1 < nch)
                def _():
                    issue(k + 1, 1 - slot)

                drain(slot)
                process(slot)
            return 0

        lax.fori_loop(0, nch // 2, pair, 0)

        for d in range(F):
            ad = acs[d]

            @plsc.parallel_loop(0, N // L, 1, unroll=4)
            def _(i):
                ad[pl.ds(i * L, L)] *= dinv_v[pl.ds(i * L, L)]

            doff = pl.multiple_of((wid * F + d) * N, N)
            pltpu.sync_copy(ad, accH.at[pl.ds(doff, N)])

    run_graph(xtl, sdl, ewl, accl)
    run_graph(xtg, sdg, ewg, accg)


_sc_propagate = pl.kernel(
    _sc_body,
    out_type=[
        jax.ShapeDtypeStruct((D * N,), jnp.float32),
        jax.ShapeDtypeStruct((D * N,), jnp.float32),
    ],
    mesh=plsc.VectorSubcoreMesh(core_axis_name="c", subcore_axis_name="s"),
    compiler_params=pltpu.CompilerParams(needs_layout_passes=False),
    scratch_types=[
        pltpu.VMEM((N,), jnp.int32),
        pltpu.VMEM((N,), jnp.int32),
        pltpu.VMEM((N,), jnp.float32),
        pltpu.VMEM((N,), jnp.float32),
        pltpu.VMEM((N,), jnp.float32),
        pltpu.VMEM((N,), jnp.float32),
        pltpu.VMEM((NP,), jnp.float32),
        pltpu.VMEM((2 * CHUNK,), jnp.int32),
        pltpu.VMEM((2 * CHUNK,), jnp.float32),
        pltpu.VMEM((2 * NPC,), jnp.float32),
        pltpu.VMEM_SHARED((NS * NS * NPC,), jnp.float32),
        pltpu.VMEM_SHARED((NP,), jnp.float32),
        pltpu.SemaphoreType.DMA((3,)),
    ],
)


def _gates(yT, Wz, bz, Lz, lzb, Wh, bh, Lh, lhb):
    Lzt = Lz[...][:D, :]
    Mz = jnp.dot(Wz[...], Lzt, preferred_element_type=jnp.float32)
    czb = jnp.dot(bz[...], Lzt, preferred_element_type=jnp.float32) + lzb[...]
    Lht = Lh[...][:D, :]
    Mh = jnp.dot(Wh[...], Lht, preferred_element_type=jnp.float32)
    chb = jnp.dot(bh[...], Lht, preferred_element_type=jnp.float32) + lhb[...]
    z = lax.dot_general(yT, Mz, (((0,), (0,)), ((), ())),
                        preferred_element_type=jnp.float32)
    h = lax.dot_general(yT, Mh, (((0,), (0,)), ((), ())),
                        preferred_element_type=jnp.float32)
    return (1.0 - jax.nn.sigmoid(z + czb)) * jnp.tanh(h + chb)


def _tc_body(ytl, ytg,
             Wzl, bzl, Lzl, lzbl, Whl, bhl, Lhl, lhbl,
             Wzg, bzg, Lzg, lzbg, Whg, bhg, Lhg, lhbg,
             outl, outg):
    outl[...] = _gates(ytl[...], Wzl, bzl, Lzl, lzbl, Whl, bhl, Lhl, lhbl)
    outg[...] = _gates(ytg[...], Wzg, bzg, Lzg, lzbg, Whg, bhg, Lhg, lhbg)


_tc_combine = pl.pallas_call(
    _tc_body,
    out_shape=[jax.ShapeDtypeStruct((N, D), jnp.float32),
               jax.ShapeDtypeStruct((N, D), jnp.float32)],
)


def _param_block(params):
    Wz, bz, Lz, lzb, _Wr, _br, _Lr, _lrb, Wh, bh, Lh, lhb = params
    return (Wz, bz.reshape(1, D), Lz, lzb.reshape(1, D),
            Wh, bh.reshape(1, D), Lh, lhb.reshape(1, D))


def kernel(local_x, global_x, local_edge_index, global_edge_index,
           local_edge_weight, global_edge_weight, local_params, global_params):
    xtl = local_x.T.reshape(-1)
    xtg = global_x.T.reshape(-1)
    sdl = jnp.bitwise_or(jnp.left_shift(local_edge_index[0], 14),
                         local_edge_index[1])
    sdg = jnp.bitwise_or(jnp.left_shift(global_edge_index[0], 14),
                         global_edge_index[1])
    ytl, ytg = _sc_propagate(xtl, sdl, local_edge_weight,
                             xtg, sdg, global_edge_weight)
    out_l, out_g = _tc_combine(
        ytl.reshape(D, N), ytg.reshape(D, N),
        *_param_block(local_params), *_param_block(global_params))
    return (out_l, out_g)
